# baseline (speedup 1.0000x reference)
import functools

import jax
import jax.numpy as jnp
from jax import lax
from jax.experimental import pallas as pl
from jax.experimental.pallas import tpu as pltpu
from jax.experimental.pallas import tpu_sc as plsc

_EMBED = 64
_LANES = 16
_NC = 2
_NS = 16
_NW = _NC * _NS
_SEQ = 50
_SEQP = 56
_SLOTS = 6
_AHEAD = 4


def _emb_body(n_groups, table_hbm, idx_hbm, out_hbm, idx_v, buf, sg, ss):
    wid = lax.axis_index("s") * _NC + lax.axis_index("c")
    base = wid * n_groups

    pltpu.sync_copy(idx_hbm.at[wid], idx_v)

    ones_i = jnp.ones((_LANES,), jnp.int32)
    zeros_i = jnp.zeros((_LANES,), jnp.int32)
    zeros_f = jnp.zeros((_LANES,), jnp.float32)

    def fire_gather(g):
        slot = lax.rem(g, _SLOTS)
        pltpu.async_copy(
            table_hbm.at[idx_v.at[g]],
            buf.at[pl.ds(slot * _SEQP, _SEQP)],
            sg,
        )

    def wait_gather(g):
        slot = lax.rem(g, _SLOTS)
        pltpu.make_async_copy(
            table_hbm.at[idx_v.at[g]],
            buf.at[pl.ds(slot * _SEQP, _SEQP)],
            sg,
        ).wait()

    def fire_scatter(g):
        slot = lax.rem(g, _SLOTS)
        pltpu.async_copy(
            buf.at[pl.ds(slot * _SEQP, _SEQP)],
            out_hbm.at[base + g, pl.ds(0, _SEQP), pl.ds(0, _EMBED)],
            ss,
        )

    def wait_scatter_one():
        pltpu.make_async_copy(
            buf.at[pl.ds(0, _SEQP)],
            out_hbm.at[base, pl.ds(0, _SEQP), pl.ds(0, _EMBED)],
            ss,
        ).wait()

    _COLS = (0, 16, 32, 40)

    def fixup(g):
        slot = lax.rem(g, _SLOTS)
        macc = zeros_i
        for c0 in _COLS:
            v = idx_v[g, pl.ds(c0, _LANES)]
            macc = macc | jnp.where(v == 0, ones_i, zeros_i)
        any_zero = macc[0]
        for l in range(1, _LANES):
            any_zero = any_zero | macc[l]

        @pl.when(any_zero != 0)
        def _():
            for c0 in _COLS:
                v = idx_v[g, pl.ds(c0, _LANES)]
                for l in range(_LANES):
                    s = v[l]

                    @pl.when(s == 0)
                    def _zero_row():
                        r = slot * _SEQP + c0 + l
                        for c in range(_EMBED // _LANES):
                            buf[r, pl.ds(c * _LANES, _LANES)] = zeros_f

    for g in range(_AHEAD):
        fire_gather(g)

    def step(g, carry):
        wait_gather(g)
        fixup(g)
        fire_scatter(g)

        @pl.when(g + _AHEAD < n_groups)
        def _():
            @pl.when(g >= _SLOTS - _AHEAD)
            def _():
                wait_scatter_one()

            fire_gather(g + _AHEAD)

        return carry

    lax.fori_loop(0, n_groups, step, 0)
    n_waited = max(0, (n_groups - _AHEAD) - (_SLOTS - _AHEAD))
    for _ in range(n_groups - n_waited):
        wait_scatter_one()


@functools.lru_cache(maxsize=None)
def _make_emb(vocab, n_groups):
    b = _NW * n_groups
    mesh = plsc.VectorSubcoreMesh(core_axis_name="c", subcore_axis_name="s")
    return pl.kernel(
        functools.partial(_emb_body, n_groups),
        mesh=mesh,
        compiler_params=pltpu.CompilerParams(use_tc_tiling_on_sc=False),
        out_type=jax.ShapeDtypeStruct((b, _SEQP, 2 * _EMBED), jnp.float32),
        scratch_types=[
            pltpu.VMEM((n_groups, _SEQP), jnp.int32),
            pltpu.VMEM((_SLOTS * _SEQP, _EMBED), jnp.float32),
            pltpu.SemaphoreType.DMA,
            pltpu.SemaphoreType.DMA,
        ],
    )


def kernel(indices, table):
    b, s = indices.shape
    n_groups = b // _NW
    idx = jnp.pad(indices.astype(jnp.int32), ((0, 0), (0, _SEQP - s)),
                  constant_values=1)
    idx = idx.reshape(_NW, n_groups, _SEQP)
    out = _make_emb(table.shape[0], n_groups)(table, idx)
    return out[:, :s, :_EMBED]
